# baseline (speedup 1.0000x reference)
import functools

import jax
import jax.numpy as jnp
from jax import lax
from jax.experimental import pallas as pl
from jax.experimental.pallas import tpu as pltpu
from jax.experimental.pallas import tpu_sc as plsc

N = 50000
E = 800000
NFEAT = 128
NHID = 128
NCLASS = 64
K = 8

NC = 2
NS = 16
LANES = 16
FH = 32

NPAD = 51200
ROWS_PER_TILE = NPAD // NS

EC = 128
CROWS = 2
CE = CROWS * EC
NCHUNKS = 198
NTRIP = NCHUNKS // 3
ROWS_PER_TILE_E = NCHUNKS * CROWS
EROWS = NS * ROWS_PER_TILE_E
EPAD = EROWS * EC
NBUF = 3
VSCALE = 2.0 ** 26


def _mlp_body(x_ref, w1_ref, b1_ref, w2_ref, b2_ref, o_ref):
    h = jnp.dot(x_ref[...], w1_ref[...], preferred_element_type=jnp.float32)
    h = jnp.maximum(h + b1_ref[...], 0.0)
    o = jnp.dot(h, w2_ref[...], preferred_element_type=jnp.float32)
    o = o + b2_ref[...]
    o_ref[0] = o[:, :FH]
    o_ref[1] = o[:, FH:]


_MLP_BM = 2048
_MLP_GRID = NPAD // _MLP_BM


def _mlp(xp, W1, b1, W2, b2):
    return pl.pallas_call(
        _mlp_body,
        grid=(_MLP_GRID,),
        in_specs=[
            pl.BlockSpec((_MLP_BM, NFEAT), lambda i: (i, 0)),
            pl.BlockSpec((NFEAT, NHID), lambda i: (0, 0)),
            pl.BlockSpec((1, NHID), lambda i: (0, 0)),
            pl.BlockSpec((NHID, NCLASS), lambda i: (0, 0)),
            pl.BlockSpec((1, NCLASS), lambda i: (0, 0)),
        ],
        out_specs=pl.BlockSpec((2, _MLP_BM, FH), lambda i: (0, i, 0)),
        out_shape=jax.ShapeDtypeStruct((2, NPAD, FH), jnp.float32),
    )(xp, W1, b1.reshape(1, NHID), W2, b2.reshape(1, NCLASS))


def _hop_body(edges_hbm, in0_hbm, in1_hbm, zeros_hbm, out_hbm,
              acc_sh, ebuf, rowsb, *sems):
    esem = sems[0:NBUF]
    gsem = sems[NBUF:2 * NBUF]
    ssem = sems[2 * NBUF:3 * NBUF]
    core = lax.axis_index("c")
    sid = lax.axis_index("s")
    erow0 = sid * ROWS_PER_TILE_E

    def fire_edges(x, t):
        pltpu.async_copy(edges_hbm.at[pl.ds(erow0 + t * CROWS, CROWS)],
                         ebuf.at[x], esem[x])

    def drain_edges(x):
        pltpu.make_async_copy(edges_hbm.at[pl.ds(erow0, CROWS)], ebuf.at[x],
                              esem[x]).wait()

    def mult_buf(x):
        for j in range(CROWS):
            def mbody(g, _, j=j):
                g16 = pl.multiple_of(g * LANES, LANES)
                v16 = ebuf[x, j, 2, pl.ds(g16, LANES)].astype(jnp.float32)
                v16 = v16 * jnp.float32(1.0 / VSCALE)
                for l in range(LANES):
                    vb = jnp.full((LANES,), v16[l], jnp.float32)
                    e = j * EC + g16 + l
                    for c in range(FH // LANES):
                        sl = pl.ds(c * LANES, LANES)
                        rowsb[x, e, sl] = rowsb[x, e, sl] * vb
                return 0
            lax.fori_loop(0, EC // LANES, mbody, 0, unroll=2)

    def run_pass(in_hbm):
        def fire_gather(x):
            for j in range(CROWS):
                pltpu.async_copy(in_hbm.at[ebuf.at[x, j, 0]],
                                 rowsb.at[x, pl.ds(j * EC, EC)], gsem[x])

        def drain_gather(x):
            for j in range(CROWS):
                pltpu.make_async_copy(in_hbm.at[ebuf.at[x, j, 0]],
                                      rowsb.at[x, pl.ds(j * EC, EC)],
                                      gsem[x]).wait()

        def fire_scatter(x):
            for j in range(CROWS):
                pltpu.async_copy(rowsb.at[x, pl.ds(j * EC, EC)],
                                 acc_sh.at[ebuf.at[x, j, 1]], ssem[x],
                                 add=True)

        def drain_scatter(x):
            for j in range(CROWS):
                pltpu.make_async_copy(rowsb.at[x, pl.ds(j * EC, EC)],
                                      acc_sh.at[ebuf.at[x, j, 1]],
                                      ssem[x]).wait()

        fire_edges(0, 0)
        fire_edges(1, 1)
        drain_edges(0)
        fire_gather(0)

        def triple(i, _):
            for s in range(NBUF):
                t = i * NBUF + s
                nxt = (s + 1) % NBUF
                prv = (s + 2) % NBUF

                @pl.when(t < NCHUNKS - 1)
                def _():
                    drain_edges(nxt)
                    fire_gather(nxt)

                drain_gather(s)
                mult_buf(s)
                fire_scatter(s)

                @pl.when(t >= 1)
                def _():
                    drain_scatter(prv)

                @pl.when(t < NCHUNKS - 2)
                def _():
                    fire_edges(prv, t + 2)
            return 0

        lax.fori_loop(0, NTRIP, triple, 0)
        drain_scatter((NCHUNKS - 1) % NBUF)

    pltpu.sync_copy(zeros_hbm,
                    acc_sh.at[pl.ds(sid * ROWS_PER_TILE, ROWS_PER_TILE)])
    plsc.subcore_barrier()

    @pl.when(core == 0)
    def _():
        run_pass(in0_hbm)

    @pl.when(core == 1)
    def _():
        run_pass(in1_hbm)

    plsc.subcore_barrier()

    pltpu.sync_copy(
        acc_sh.at[pl.ds(sid * ROWS_PER_TILE, ROWS_PER_TILE)],
        out_hbm.at[core, pl.ds(sid * ROWS_PER_TILE, ROWS_PER_TILE)])


_hop = functools.partial(
    pl.kernel,
    out_type=jax.ShapeDtypeStruct((2, NPAD, FH), jnp.float32),
    mesh=plsc.VectorSubcoreMesh(core_axis_name="c", subcore_axis_name="s"),
    compiler_params=pltpu.CompilerParams(use_tc_tiling_on_sc=False),
    scratch_types=[
        pltpu.VMEM_SHARED((NPAD, FH), jnp.float32),
        pltpu.VMEM((NBUF, CROWS, 3, EC), jnp.int32),
        pltpu.VMEM((NBUF, CE, FH), jnp.float32),
    ] + [pltpu.SemaphoreType.DMA] * (3 * NBUF),
)(_hop_body)


def kernel(x, adj_values, W1, b1, W2, b2, adj_indices):
    dst = adj_indices[0]
    src = adj_indices[1]

    epad = EPAD - E
    val_fix = (adj_values * jnp.float32(VSCALE)).astype(jnp.int32)
    srcp = jnp.concatenate([src, jnp.zeros((epad,), jnp.int32)]).reshape(EROWS, EC)
    dstp = jnp.concatenate([dst, jnp.zeros((epad,), jnp.int32)]).reshape(EROWS, EC)
    valp = jnp.concatenate([val_fix, jnp.zeros((epad,), jnp.int32)]).reshape(EROWS, EC)
    edges = jnp.stack([srcp, dstp, valp], axis=1)

    xp = jnp.pad(x, ((0, NPAD - N), (0, 0)))
    zeros_blob = jnp.zeros((ROWS_PER_TILE, FH), jnp.float32)

    h = _mlp(xp, W1, b1, W2, b2)
    for _ in range(K):
        h = _hop(edges, h[0], h[1], zeros_blob)
    return jnp.concatenate([h[0, :N], h[1, :N]], axis=1)


# speedup vs baseline: 1.0786x; 1.0786x over previous
import functools

import jax
import jax.numpy as jnp
from jax import lax
from jax.experimental import pallas as pl
from jax.experimental.pallas import tpu as pltpu
from jax.experimental.pallas import tpu_sc as plsc

N = 50000
E = 800000
NFEAT = 128
NHID = 128
NCLASS = 64
K = 8

NC = 2
NS = 16
LANES = 16
FH = 32

NPAD = 51200
ROWS_PER_TILE = NPAD // NS

EC = 128
CROWS = 2
CE = CROWS * EC
NCHUNKS = 198
NTRIP = NCHUNKS // 3
ROWS_PER_TILE_E = NCHUNKS * CROWS
EROWS = NS * ROWS_PER_TILE_E
EPAD = EROWS * EC
NBUF = 3
VSCALE = 2.0 ** 26


def _mlp_body(x_ref, w1_ref, b1_ref, w2_ref, b2_ref, o_ref):
    h = jnp.dot(x_ref[...], w1_ref[...], preferred_element_type=jnp.float32)
    h = jnp.maximum(h + b1_ref[...], 0.0)
    o = jnp.dot(h, w2_ref[...], preferred_element_type=jnp.float32)
    o = o + b2_ref[...]
    o_ref[0] = o[:, :FH]
    o_ref[1] = o[:, FH:]


_MLP_BM = 2048
_MLP_GRID = NPAD // _MLP_BM


def _mlp(xp, W1, b1, W2, b2):
    return pl.pallas_call(
        _mlp_body,
        grid=(_MLP_GRID,),
        in_specs=[
            pl.BlockSpec((_MLP_BM, NFEAT), lambda i: (i, 0)),
            pl.BlockSpec((NFEAT, NHID), lambda i: (0, 0)),
            pl.BlockSpec((1, NHID), lambda i: (0, 0)),
            pl.BlockSpec((NHID, NCLASS), lambda i: (0, 0)),
            pl.BlockSpec((1, NCLASS), lambda i: (0, 0)),
        ],
        out_specs=pl.BlockSpec((2, _MLP_BM, FH), lambda i: (0, i, 0)),
        out_shape=jax.ShapeDtypeStruct((2, NPAD, FH), jnp.float32),
    )(xp, W1, b1.reshape(1, NHID), W2, b2.reshape(1, NCLASS))


def _hop_body(edges_hbm, in0_hbm, in1_hbm, zeros_hbm, out_hbm,
              acc_sh, ebuf, rowsb, *sems):
    esem = sems[0:NBUF]
    gsem = sems[NBUF:2 * NBUF]
    ssem = sems[2 * NBUF:3 * NBUF]
    core = lax.axis_index("c")
    sid = lax.axis_index("s")
    erow0 = sid * ROWS_PER_TILE_E

    def fire_edges(x, t):
        pltpu.async_copy(edges_hbm.at[pl.ds(erow0 + t * CROWS, CROWS)],
                         ebuf.at[x], esem[x])

    def drain_edges(x):
        pltpu.make_async_copy(edges_hbm.at[pl.ds(erow0, CROWS)], ebuf.at[x],
                              esem[x]).wait()

    def mult_buf(x):
        for j in range(CROWS):
            def mbody(g, _, j=j):
                g16 = pl.multiple_of(g * LANES, LANES)
                v16 = ebuf[x, j, 2, pl.ds(g16, LANES)].astype(jnp.float32)
                v16 = v16 * jnp.float32(1.0 / VSCALE)
                for l in range(LANES):
                    vb = jnp.full((LANES,), v16[l], jnp.float32)
                    e = j * EC + g16 + l
                    for c in range(FH // LANES):
                        sl = pl.ds(c * LANES, LANES)
                        rowsb[x, e, sl] = rowsb[x, e, sl] * vb
                return 0
            lax.fori_loop(0, EC // LANES, mbody, 0)

    def run_pass(in_hbm):
        def fire_gather(x):
            for j in range(CROWS):
                pltpu.async_copy(in_hbm.at[ebuf.at[x, j, 0]],
                                 rowsb.at[x, pl.ds(j * EC, EC)], gsem[x])

        def drain_gather(x):
            for j in range(CROWS):
                pltpu.make_async_copy(in_hbm.at[ebuf.at[x, j, 0]],
                                      rowsb.at[x, pl.ds(j * EC, EC)],
                                      gsem[x]).wait()

        def fire_scatter(x):
            for j in range(CROWS):
                pltpu.async_copy(rowsb.at[x, pl.ds(j * EC, EC)],
                                 acc_sh.at[ebuf.at[x, j, 1]], ssem[x],
                                 add=True)

        def drain_scatter(x):
            for j in range(CROWS):
                pltpu.make_async_copy(rowsb.at[x, pl.ds(j * EC, EC)],
                                      acc_sh.at[ebuf.at[x, j, 1]],
                                      ssem[x]).wait()

        fire_edges(0, 0)
        fire_edges(1, 1)
        drain_edges(0)
        fire_gather(0)

        def triple(i, _):
            for s in range(NBUF):
                t = i * NBUF + s
                nxt = (s + 1) % NBUF
                prv = (s + 2) % NBUF

                @pl.when(t < NCHUNKS - 1)
                def _():
                    drain_edges(nxt)
                    fire_gather(nxt)

                drain_gather(s)
                mult_buf(s)
                fire_scatter(s)

                @pl.when(t >= 1)
                def _():
                    drain_scatter(prv)

                @pl.when(t < NCHUNKS - 2)
                def _():
                    fire_edges(prv, t + 2)
            return 0

        lax.fori_loop(0, NTRIP, triple, 0)
        drain_scatter((NCHUNKS - 1) % NBUF)

    pltpu.sync_copy(zeros_hbm,
                    acc_sh.at[pl.ds(sid * ROWS_PER_TILE, ROWS_PER_TILE)])
    plsc.subcore_barrier()

    @pl.when(core == 0)
    def _():
        run_pass(in0_hbm)

    @pl.when(core == 1)
    def _():
        run_pass(in1_hbm)

    plsc.subcore_barrier()

    pltpu.sync_copy(
        acc_sh.at[pl.ds(sid * ROWS_PER_TILE, ROWS_PER_TILE)],
        out_hbm.at[core, pl.ds(sid * ROWS_PER_TILE, ROWS_PER_TILE)])


_hop = functools.partial(
    pl.kernel,
    out_type=jax.ShapeDtypeStruct((2, NPAD, FH), jnp.float32),
    mesh=plsc.VectorSubcoreMesh(core_axis_name="c", subcore_axis_name="s"),
    compiler_params=pltpu.CompilerParams(use_tc_tiling_on_sc=False),
    scratch_types=[
        pltpu.VMEM_SHARED((NPAD, FH), jnp.float32),
        pltpu.VMEM((NBUF, CROWS, 3, EC), jnp.int32),
        pltpu.VMEM((NBUF, CE, FH), jnp.float32),
    ] + [pltpu.SemaphoreType.DMA] * (3 * NBUF),
)(_hop_body)


def kernel(x, adj_values, W1, b1, W2, b2, adj_indices):
    dst = adj_indices[0]
    src = adj_indices[1]

    epad = EPAD - E
    val_fix = (adj_values * jnp.float32(VSCALE)).astype(jnp.int32)
    srcp = jnp.concatenate([src, jnp.zeros((epad,), jnp.int32)]).reshape(EROWS, EC)
    dstp = jnp.concatenate([dst, jnp.zeros((epad,), jnp.int32)]).reshape(EROWS, EC)
    valp = jnp.concatenate([val_fix, jnp.zeros((epad,), jnp.int32)]).reshape(EROWS, EC)
    edges = jnp.stack([srcp, dstp, valp], axis=1)

    xp = jnp.pad(x, ((0, NPAD - N), (0, 0)))
    zeros_blob = jnp.zeros((ROWS_PER_TILE, FH), jnp.float32)

    h = _mlp(xp, W1, b1, W2, b2)
    for _ in range(K):
        h = _hop(edges, h[0], h[1], zeros_blob)
    return jnp.concatenate([h[0, :N], h[1, :N]], axis=1)


# speedup vs baseline: 1.3529x; 1.2542x over previous
import functools

import jax
import jax.numpy as jnp
from jax import lax
from jax.experimental import pallas as pl
from jax.experimental.pallas import tpu as pltpu
from jax.experimental.pallas import tpu_sc as plsc

N = 50000
E = 800000
NFEAT = 128
NHID = 128
NCLASS = 64
K = 8

NC = 2
NS = 16
LANES = 16
FH = 32

NPAD = 51200
ROWS_PER_TILE = NPAD // NS

EC = 128
CROWS = 2
CE = CROWS * EC
NCHUNKS = 198
NTRIP = NCHUNKS // 3
ROWS_PER_TILE_E = NCHUNKS * CROWS
EROWS = NS * ROWS_PER_TILE_E
EPAD = EROWS * EC
NBUF = 3
VSCALE = 2.0 ** 26


def _mlp_body(x_ref, w1_ref, b1_ref, w2_ref, b2_ref, o_ref):
    h = jnp.dot(x_ref[...], w1_ref[...], preferred_element_type=jnp.float32)
    h = jnp.maximum(h + b1_ref[...], 0.0)
    o = jnp.dot(h, w2_ref[...], preferred_element_type=jnp.float32)
    o = o + b2_ref[...]
    o_ref[0] = o[:, :FH]
    o_ref[1] = o[:, FH:]


_MLP_BM = 2000
_MLP_GRID = N // _MLP_BM


def _mlp(x, W1, b1, W2, b2):
    return pl.pallas_call(
        _mlp_body,
        grid=(_MLP_GRID,),
        in_specs=[
            pl.BlockSpec((_MLP_BM, NFEAT), lambda i: (i, 0)),
            pl.BlockSpec((NFEAT, NHID), lambda i: (0, 0)),
            pl.BlockSpec((1, NHID), lambda i: (0, 0)),
            pl.BlockSpec((NHID, NCLASS), lambda i: (0, 0)),
            pl.BlockSpec((1, NCLASS), lambda i: (0, 0)),
        ],
        out_specs=pl.BlockSpec((2, _MLP_BM, FH), lambda i: (0, i, 0)),
        out_shape=jax.ShapeDtypeStruct((2, NPAD, FH), jnp.float32),
    )(x, W1, b1.reshape(1, NHID), W2, b2.reshape(1, NCLASS))


def _hop_body(edges_hbm, in_hbm, zeros_hbm, out_hbm,
              acc_sh, ebuf, rowsb, *sems):
    esem = sems[0:NBUF]
    gsem = sems[NBUF:2 * NBUF]
    ssem = sems[2 * NBUF:3 * NBUF]
    core = lax.axis_index("c")
    sid = lax.axis_index("s")
    erow0 = sid * ROWS_PER_TILE_E

    def fire_edges(x, t):
        pltpu.async_copy(edges_hbm.at[pl.ds(erow0 + t * CROWS, CROWS)],
                         ebuf.at[x], esem[x])

    def drain_edges(x):
        pltpu.make_async_copy(edges_hbm.at[pl.ds(erow0, CROWS)], ebuf.at[x],
                              esem[x]).wait()

    def adjust_src(x):
        off = core * NPAD
        for j in range(CROWS):
            for g in range(EC // LANES):
                sl = pl.ds(g * LANES, LANES)
                ebuf[x, j, 0, sl] = ebuf[x, j, 0, sl] + off

    def mult_buf(x):
        for j in range(CROWS):
            def mbody(g, _, j=j):
                g16 = pl.multiple_of(g * LANES, LANES)
                v16 = ebuf[x, j, 2, pl.ds(g16, LANES)].astype(jnp.float32)
                v16 = v16 * jnp.float32(1.0 / VSCALE)
                for l in range(LANES):
                    vb = jnp.full((LANES,), v16[l], jnp.float32)
                    e = j * EC + g16 + l
                    for c in range(FH // LANES):
                        sl = pl.ds(c * LANES, LANES)
                        rowsb[x, e, sl] = rowsb[x, e, sl] * vb
                return 0
            lax.fori_loop(0, EC // LANES, mbody, 0)

    def run_pass():
        def fire_gather(x):
            for j in range(CROWS):
                pltpu.async_copy(in_hbm.at[ebuf.at[x, j, 0]],
                                 rowsb.at[x, pl.ds(j * EC, EC)], gsem[x])

        def drain_gather(x):
            for j in range(CROWS):
                pltpu.make_async_copy(in_hbm.at[ebuf.at[x, j, 0]],
                                      rowsb.at[x, pl.ds(j * EC, EC)],
                                      gsem[x]).wait()

        def fire_scatter(x):
            for j in range(CROWS):
                pltpu.async_copy(rowsb.at[x, pl.ds(j * EC, EC)],
                                 acc_sh.at[ebuf.at[x, j, 1]], ssem[x],
                                 add=True)

        def drain_scatter(x):
            for j in range(CROWS):
                pltpu.make_async_copy(rowsb.at[x, pl.ds(j * EC, EC)],
                                      acc_sh.at[ebuf.at[x, j, 1]],
                                      ssem[x]).wait()

        fire_edges(0, 0)
        fire_edges(1, 1)
        drain_edges(0)
        adjust_src(0)
        fire_gather(0)

        def triple(i, _):
            for s in range(NBUF):
                t = i * NBUF + s
                nxt = (s + 1) % NBUF
                prv = (s + 2) % NBUF

                @pl.when(t < NCHUNKS - 1)
                def _():
                    drain_edges(nxt)
                    adjust_src(nxt)
                    fire_gather(nxt)

                drain_gather(s)
                mult_buf(s)
                fire_scatter(s)

                @pl.when(t >= 1)
                def _():
                    drain_scatter(prv)

                @pl.when(t < NCHUNKS - 2)
                def _():
                    fire_edges(prv, t + 2)
            return 0

        lax.fori_loop(0, NTRIP, triple, 0)
        drain_scatter((NCHUNKS - 1) % NBUF)

    pltpu.sync_copy(zeros_hbm,
                    acc_sh.at[pl.ds(sid * ROWS_PER_TILE, ROWS_PER_TILE)])
    plsc.subcore_barrier()

    run_pass()

    plsc.subcore_barrier()

    pltpu.sync_copy(
        acc_sh.at[pl.ds(sid * ROWS_PER_TILE, ROWS_PER_TILE)],
        out_hbm.at[core, pl.ds(sid * ROWS_PER_TILE, ROWS_PER_TILE)])


_hop = functools.partial(
    pl.kernel,
    out_type=jax.ShapeDtypeStruct((2, NPAD, FH), jnp.float32),
    mesh=plsc.VectorSubcoreMesh(core_axis_name="c", subcore_axis_name="s"),
    compiler_params=pltpu.CompilerParams(use_tc_tiling_on_sc=False),
    scratch_types=[
        pltpu.VMEM_SHARED((NPAD, FH), jnp.float32),
        pltpu.VMEM((NBUF, CROWS, 3, EC), jnp.int32),
        pltpu.VMEM((NBUF, CE, FH), jnp.float32),
    ] + [pltpu.SemaphoreType.DMA] * (3 * NBUF),
)(_hop_body)


def kernel(x, adj_values, W1, b1, W2, b2, adj_indices):
    dst = adj_indices[0]
    src = adj_indices[1]

    epad = EPAD - E
    val_fix = (adj_values * jnp.float32(VSCALE)).astype(jnp.int32)
    srcp = jnp.concatenate([src, jnp.zeros((epad,), jnp.int32)]).reshape(EROWS, EC)
    dstp = jnp.concatenate([dst, jnp.zeros((epad,), jnp.int32)]).reshape(EROWS, EC)
    valp = jnp.concatenate([val_fix, jnp.zeros((epad,), jnp.int32)]).reshape(EROWS, EC)
    edges = jnp.stack([srcp, dstp, valp], axis=1)

    zeros_blob = jnp.zeros((ROWS_PER_TILE, FH), jnp.float32)

    h = _mlp(x, W1, b1, W2, b2)
    for _ in range(K):
        h = _hop(edges, h.reshape(2 * NPAD, FH), zeros_blob)
    return jnp.concatenate([h[0, :N], h[1, :N]], axis=1)
